# baseline (speedup 1.0000x reference)
import functools

import jax
import jax.numpy as jnp
from jax import lax
from jax.experimental import pallas as pl
from jax.experimental.pallas import tpu as pltpu
from jax.experimental.pallas import tpu_sc as plsc

NHID = 128
NG = 64
N = 10000
E = 320000
NWORK = 32
SHARD = E // NWORK
FPT = NHID // NWORK
WIN = 2000

_SC_MESH = plsc.VectorSubcoreMesh(core_axis_name="c", subcore_axis_name="s",
                                  num_cores=2, num_subcores=16)
_SC_PARAMS = pltpu.CompilerParams(needs_layout_passes=False)


def _wid():
    return lax.axis_index("s") * 2 + lax.axis_index("c")


def _gcn(x, W, b, src, dst, w):
    n = x.shape[0]
    xw = x @ W
    deg = jnp.zeros((n,), x.dtype).at[dst].add(w) + 1.0
    norm = w / jnp.sqrt(deg[src] * deg[dst])
    agg = jnp.zeros_like(xw).at[dst].add(xw[src] * norm[:, None])
    agg = agg + xw / deg[:, None]
    return agg + b


def _tc_pre_kernel(degp_ref, xgT_ref, WT_ref, xwT_ref, isd_ref, invdeg_ref):
    deg = jnp.sum(degp_ref[...], axis=0, keepdims=True) + 1.0
    isd_ref[...] = lax.rsqrt(deg)
    invdeg_ref[...] = 1.0 / deg
    xwT_ref[...] = jnp.dot(WT_ref[...], xgT_ref[...],
                           preferred_element_type=jnp.float32)


def _tc_pre(degp, xgT, WT):
    return pl.pallas_call(
        _tc_pre_kernel,
        name="tc_pre",
        out_shape=(
            jax.ShapeDtypeStruct((NHID, N), jnp.float32),
            jax.ShapeDtypeStruct((1, N), jnp.float32),
            jax.ShapeDtypeStruct((1, N), jnp.float32),
        ),
    )(degp, xgT, WT)


def _tc_postagg_kernel(aggT_ref, xwT_ref, invdeg_ref, bcol_ref, WpT_ref,
                       xT_ref, xws_ref):
    xT = jnp.maximum(
        aggT_ref[...] + xwT_ref[...] * invdeg_ref[...] + bcol_ref[...], 0.0)
    xT_ref[...] = xT
    xws_ref[...] = jnp.dot(WpT_ref[...], xT,
                           preferred_element_type=jnp.float32)


def _tc_postagg(aggT, xwT, invdeg, bcol, WpT):
    return pl.pallas_call(
        _tc_postagg_kernel,
        name="tc_postagg",
        out_shape=(
            jax.ShapeDtypeStruct((NHID, N), jnp.float32),
            jax.ShapeDtypeStruct((1, N), jnp.float32),
        ),
    )(aggT, xwT, invdeg, bcol, WpT)


def _postscore_body(s, mask_in_ref, xT_ref, batch_col_ref, batch_row_ref,
                    k, n_level, mask_ref, xgT_ref, xrT_ref, kl_ref):
    kept = mask_in_ref[...] > 0.0

    neg_inf = jnp.float32(-jnp.inf)
    smax = jnp.max(jnp.where(kept, s, neg_inf))
    e = jnp.where(kept, jnp.exp(s - smax), 0.0)
    p = e / jnp.sum(e)
    term = p * (jnp.log(p + 1e-12) + jnp.log(jnp.float32(n_level)))
    kl_ref[...] = jnp.sum(jnp.where(kept, term, 0.0)).reshape(1, 1)

    ubits = lax.bitcast_convert_type(s, jnp.uint32)
    u = jnp.where(s < 0.0, ~ubits, ubits | jnp.uint32(0x80000000))
    u = jnp.where(kept, u, jnp.uint32(0))
    kf = jnp.float32(k)

    def sel_bit(i, P):
        cand = P | (jnp.uint32(1) << (jnp.uint32(31) - i.astype(jnp.uint32)))
        c = jnp.sum((u >= cand).astype(jnp.float32))
        return jnp.where(c >= kf, cand, P)

    P = lax.fori_loop(0, 32, sel_bit, jnp.uint32(0))
    c_strict = jnp.sum((u > P).astype(jnp.float32))
    need = kf - c_strict
    tie = kept & (u == P)
    iota = lax.broadcasted_iota(jnp.int32, (1, N), 1)

    def idx_bit(i, t):
        cand = t + (jnp.int32(1) << (jnp.int32(13) - i))
        c = jnp.sum((tie & (iota < cand)).astype(jnp.float32))
        return jnp.where(c < need, cand, t)

    t_acc = lax.fori_loop(0, 14, idx_bit, jnp.int32(0))
    sel = (u > P) | (tie & (iota <= t_acc) & (need > 0.5))
    mask_out = jnp.where(sel, 1.0, 0.0)
    mask_ref[...] = mask_out

    gate = jnp.tanh(s) * mask_out
    xgT = xT_ref[...] * gate
    xgT_ref[...] = xgT

    B = (batch_col_ref[...] == lax.broadcasted_iota(
        jnp.int32, (N, NG), 1)).astype(jnp.float32)
    ssumT = jnp.dot(xgT, B, preferred_element_type=jnp.float32)
    cnt = jnp.dot(mask_out, B, preferred_element_type=jnp.float32)
    meanT = ssumT / jnp.maximum(cnt, 1.0)

    batch_row = batch_row_ref[...]
    col_iota = lax.broadcasted_iota(jnp.int32, (1, NG), 1)

    def mx_body(g, mxT):
        selg = (batch_row == g) & sel
        cand = jnp.max(jnp.where(selg, xgT, neg_inf), axis=1, keepdims=True)
        cand_b = jnp.broadcast_to(cand, (NHID, NG))
        return jnp.where(col_iota == g, cand_b, mxT)

    mxT = lax.fori_loop(0, NG, mx_body, jnp.full((NHID, NG), neg_inf))
    mxT = jnp.where(mxT == neg_inf, 0.0, mxT)
    xrT_ref[...] = jnp.concatenate([mxT, meanT], axis=0)


def _tc_postscore1_kernel(s_ref, mask_in_ref, xT_ref, batch_col_ref,
                          batch_row_ref, mask_ref, xgT_ref, xrT_ref, kl_ref,
                          *, k, n_level):
    _postscore_body(s_ref[...], mask_in_ref, xT_ref, batch_col_ref,
                    batch_row_ref, k, n_level, mask_ref, xgT_ref, xrT_ref,
                    kl_ref)


def _tc_postscore23_kernel(aggsp_ref, xws_ref, invdeg_ref, bp_ref, mask_in_ref,
                           xT_ref, batch_col_ref, batch_row_ref, mask_ref,
                           xgT_ref, xrT_ref, kl_ref, *, k, n_level):
    s = (jnp.sum(aggsp_ref[...], axis=0, keepdims=True)
         + xws_ref[...] * invdeg_ref[...] + bp_ref[0, 0])
    _postscore_body(s, mask_in_ref, xT_ref, batch_col_ref, batch_row_ref,
                    k, n_level, mask_ref, xgT_ref, xrT_ref, kl_ref)


_POSTSCORE_OUT = (
    jax.ShapeDtypeStruct((1, N), jnp.float32),
    jax.ShapeDtypeStruct((NHID, N), jnp.float32),
    jax.ShapeDtypeStruct((2 * NHID, NG), jnp.float32),
    jax.ShapeDtypeStruct((1, 1), jnp.float32),
)


def _tc_postscore1(s, mask_in, xT, batch_col, batch_row, k, n_level):
    return pl.pallas_call(
        functools.partial(_tc_postscore1_kernel, k=k, n_level=n_level),
        name="tc_postscore1",
        out_shape=_POSTSCORE_OUT,
    )(s, mask_in, xT, batch_col, batch_row)


def _tc_postscore23(aggsp, xws, invdeg, bp, mask_in, xT, batch_col, batch_row,
                    k, n_level):
    return pl.pallas_call(
        functools.partial(_tc_postscore23_kernel, k=k, n_level=n_level),
        name="tc_postscore23",
        out_shape=_POSTSCORE_OUT,
    )(aggsp, xws, invdeg, bp, mask_in, xT, batch_col, batch_row)


def _tc_head_kernel(x1_ref, x2_ref, x3_ref, lab_ref, W1T_ref, b1c_ref,
                    W2T_ref, b2c_ref, W3T_ref, b3_ref, out_ref, loss_ref):
    xrT = x1_ref[...] + x2_ref[...] + x3_ref[...]
    h = jnp.maximum(
        jnp.dot(W1T_ref[...], xrT, preferred_element_type=jnp.float32)
        + b1c_ref[...], 0.0)
    h = jnp.maximum(
        jnp.dot(W2T_ref[...], h, preferred_element_type=jnp.float32)
        + b2c_ref[...], 0.0)
    outT = jnp.dot(W3T_ref[...], h,
                   preferred_element_type=jnp.float32) + b3_ref[0, 0]
    out_ref[...] = outT
    loss_ref[...] = jnp.mean(jnp.abs(outT - lab_ref[...])).reshape(1, 1)


def _tc_head(xrT1, xrT2, xrT3, labT, Wl1T, bl1c, Wl2T, bl2c, Wl3T, bl3):
    return pl.pallas_call(
        _tc_head_kernel,
        name="tc_head",
        out_shape=(
            jax.ShapeDtypeStruct((1, NG), jnp.float32),
            jax.ShapeDtypeStruct((1, 1), jnp.float32),
        ),
    )(xrT1, xrT2, xrT3, labT, Wl1T, bl1c, Wl2T, bl2c, Wl3T, bl3)


def _zero_vmem(ref, n):
    def body(i, _):
        ref[pl.ds(i * 16, 16)] = jnp.zeros((16,), jnp.float32)
        return 0
    lax.fori_loop(0, n // 16, body, 0)


def _sc_prep_body(src_hbm, dst_hbm, w_hbm, mask_hbm, wout_hbm, degp_hbm,
                  srcv, dstv, wv, maskv, woutv, accv):
    wid = _wid()
    base = wid * SHARD
    pltpu.sync_copy(src_hbm.at[pl.ds(base, SHARD)], srcv)
    pltpu.sync_copy(dst_hbm.at[pl.ds(base, SHARD)], dstv)
    pltpu.sync_copy(w_hbm.at[pl.ds(base, SHARD)], wv)
    pltpu.sync_copy(mask_hbm, maskv)
    _zero_vmem(accv, N)

    @plsc.parallel_loop(0, SHARD // 16, unroll=4)
    def edge(g):
        sl = pl.ds(g * 16, 16)
        s16, d16, w16 = srcv[sl], dstv[sl], wv[sl]
        wo = (w16 * plsc.load_gather(maskv, [s16])
              * plsc.load_gather(maskv, [d16]))
        woutv[sl] = wo
        plsc.addupdate_scatter(accv, [d16], wo)
    pltpu.sync_copy(woutv, wout_hbm.at[pl.ds(base, SHARD)])
    pltpu.sync_copy(accv, degp_hbm.at[wid])


_sc_prep = functools.partial(
    pl.kernel, _sc_prep_body, name="sc_prep", mesh=_SC_MESH, compiler_params=_SC_PARAMS,
    out_type=(
        jax.ShapeDtypeStruct((E,), jnp.float32),
        jax.ShapeDtypeStruct((NWORK, N), jnp.float32),
    ),
    scratch_types=[
        pltpu.VMEM((SHARD,), jnp.int32),
        pltpu.VMEM((SHARD,), jnp.int32),
        pltpu.VMEM((SHARD,), jnp.float32),
        pltpu.VMEM((N,), jnp.float32),
        pltpu.VMEM((SHARD,), jnp.float32),
        pltpu.VMEM((N,), jnp.float32),
    ],
)()


def _sc_score_body(src_hbm, dst_hbm, w_hbm, isd_hbm, xws_hbm, aggsp_hbm,
                   srcv, dstv, wv, isdv, xwsv, accv):
    wid = _wid()
    base = wid * SHARD
    pltpu.sync_copy(src_hbm.at[pl.ds(base, SHARD)], srcv)
    pltpu.sync_copy(dst_hbm.at[pl.ds(base, SHARD)], dstv)
    pltpu.sync_copy(w_hbm.at[pl.ds(base, SHARD)], wv)
    pltpu.sync_copy(isd_hbm, isdv)
    pltpu.sync_copy(xws_hbm, xwsv)
    _zero_vmem(accv, N)

    @plsc.parallel_loop(0, SHARD // 16, unroll=4)
    def edge(g):
        sl = pl.ds(g * 16, 16)
        s16, d16, w16 = srcv[sl], dstv[sl], wv[sl]
        n16 = (w16 * plsc.load_gather(isdv, [s16])
               * plsc.load_gather(isdv, [d16]))
        val = plsc.load_gather(xwsv, [s16]) * n16
        plsc.addupdate_scatter(accv, [d16], val)
    pltpu.sync_copy(accv, aggsp_hbm.at[wid])


_sc_score = functools.partial(
    pl.kernel, _sc_score_body, name="sc_score", mesh=_SC_MESH, compiler_params=_SC_PARAMS,
    out_type=jax.ShapeDtypeStruct((NWORK, N), jnp.float32),
    scratch_types=[
        pltpu.VMEM((SHARD,), jnp.int32),
        pltpu.VMEM((SHARD,), jnp.int32),
        pltpu.VMEM((SHARD,), jnp.float32),
        pltpu.VMEM((N,), jnp.float32),
        pltpu.VMEM((N,), jnp.float32),
        pltpu.VMEM((N,), jnp.float32),
    ],
)()


def _sc_heavy_body(src_hbm, dst_hbm, w_hbm, isd_hbm, xwTf_hbm, aggTf_hbm,
                   srcwv, dstwv, wwv, isdv, *frefs):
    xwfs, accfs = frefs[:FPT], frefs[FPT:]
    wid = _wid()
    f0 = wid * FPT
    for f in range(FPT):
        pltpu.sync_copy(xwTf_hbm.at[pl.ds((f0 + f) * N, N)], xwfs[f])
        _zero_vmem(accfs[f], N)
    pltpu.sync_copy(isd_hbm, isdv)

    def window(wi, _):
        b = wi * WIN
        pltpu.sync_copy(src_hbm.at[pl.ds(b, WIN)], srcwv)
        pltpu.sync_copy(dst_hbm.at[pl.ds(b, WIN)], dstwv)
        pltpu.sync_copy(w_hbm.at[pl.ds(b, WIN)], wwv)

        @plsc.parallel_loop(0, WIN // 16, unroll=4)
        def edge(g):
            sl = pl.ds(g * 16, 16)
            s16, d16, w16 = srcwv[sl], dstwv[sl], wwv[sl]
            n16 = (w16 * plsc.load_gather(isdv, [s16])
                   * plsc.load_gather(isdv, [d16]))
            for f in range(FPT):
                v = plsc.load_gather(xwfs[f], [s16]) * n16
                plsc.addupdate_scatter(accfs[f], [d16], v)

        return 0

    lax.fori_loop(0, E // WIN, window, 0)
    for f in range(FPT):
        pltpu.sync_copy(accfs[f], aggTf_hbm.at[pl.ds((f0 + f) * N, N)])


_sc_heavy_flat = functools.partial(
    pl.kernel, _sc_heavy_body, name="sc_heavy", mesh=_SC_MESH, compiler_params=_SC_PARAMS,
    out_type=jax.ShapeDtypeStruct((NHID * N,), jnp.float32),
    scratch_types=[
        pltpu.VMEM((WIN,), jnp.int32),
        pltpu.VMEM((WIN,), jnp.int32),
        pltpu.VMEM((WIN,), jnp.float32),
        pltpu.VMEM((N,), jnp.float32),
    ] + [pltpu.VMEM((N,), jnp.float32)] * (2 * FPT),
)()


def _sc_heavy(src, dst, w, isd, xwT):
    aggTf = _sc_heavy_flat(src, dst, w, isd, xwT.reshape(NHID * N))
    return aggTf.reshape(NHID, N)


CH = 80


def _sc_degprod_body(src_hbm, dst_hbm, deg_hbm, out_hbm, srcv, dstv, degv,
                     outv):
    wid = _wid()
    base = wid * SHARD
    pltpu.sync_copy(src_hbm.at[pl.ds(base, SHARD)], srcv)
    pltpu.sync_copy(dst_hbm.at[pl.ds(base, SHARD)], dstv)
    pltpu.sync_copy(deg_hbm, degv)

    @plsc.parallel_loop(0, SHARD // 16, unroll=4)
    def edge(g):
        sl = pl.ds(g * 16, 16)
        outv[sl] = (plsc.load_gather(degv, [srcv[sl]])
                    * plsc.load_gather(degv, [dstv[sl]]))
    pltpu.sync_copy(outv, out_hbm.at[pl.ds(base, SHARD)])


_sc_degprod = functools.partial(
    pl.kernel, _sc_degprod_body, name="sc_degprod", mesh=_SC_MESH, compiler_params=_SC_PARAMS,
    out_type=jax.ShapeDtypeStruct((E,), jnp.float32),
    scratch_types=[
        pltpu.VMEM((SHARD,), jnp.int32),
        pltpu.VMEM((SHARD,), jnp.int32),
        pltpu.VMEM((N,), jnp.float32),
        pltpu.VMEM((SHARD,), jnp.float32),
    ],
)()


def _sc_smul_body(src_hbm, xws_hbm, norm_hbm, out_hbm, srcv, xwsv, normv,
                  outv):
    wid = _wid()
    base = wid * SHARD
    pltpu.sync_copy(src_hbm.at[pl.ds(base, SHARD)], srcv)
    pltpu.sync_copy(norm_hbm.at[pl.ds(base, SHARD)], normv)
    pltpu.sync_copy(xws_hbm, xwsv)

    @plsc.parallel_loop(0, SHARD // 16, unroll=4)
    def edge(g):
        sl = pl.ds(g * 16, 16)
        outv[sl] = plsc.load_gather(xwsv, [srcv[sl]]) * normv[sl]
    pltpu.sync_copy(outv, out_hbm.at[pl.ds(base, SHARD)])


_sc_smul = functools.partial(
    pl.kernel, _sc_smul_body, name="sc_smul", mesh=_SC_MESH, compiler_params=_SC_PARAMS,
    out_type=jax.ShapeDtypeStruct((E,), jnp.float32),
    scratch_types=[
        pltpu.VMEM((SHARD,), jnp.int32),
        pltpu.VMEM((N,), jnp.float32),
        pltpu.VMEM((SHARD,), jnp.float32),
        pltpu.VMEM((SHARD,), jnp.float32),
    ],
)()


def _sc_rowgather_body(src_hbm, xw_hbm, out_hbm, srcc, rows, sem):
    wid = _wid()
    base = wid * SHARD

    def chunk(c, _):
        b = base + c * CH
        pltpu.sync_copy(src_hbm.at[pl.ds(b, CH)], srcc)
        pltpu.async_copy(xw_hbm.at[srcc], rows, sem).wait()
        pltpu.sync_copy(rows, out_hbm.at[pl.ds(b, CH)])
        return 0

    lax.fori_loop(0, SHARD // CH, chunk, 0)


_sc_rowgather = functools.partial(
    pl.kernel, _sc_rowgather_body, name="sc_rowgather", mesh=_SC_MESH, compiler_params=_SC_PARAMS,
    out_type=jax.ShapeDtypeStruct((E, NHID), jnp.float32),
    scratch_types=[
        pltpu.VMEM((CH,), jnp.int32),
        pltpu.VMEM((CH, NHID), jnp.float32),
        pltpu.SemaphoreType.DMA,
    ],
)()


def _level23(src, dst, w_in, mask_in, xgT, WT, bcol, WpT, bp, batch_col,
             batch_row, k, n_level):
    w_out, degp = _sc_prep(src, dst, w_in, mask_in.reshape(N))
    xwT, isd, invdeg = _tc_pre(degp, xgT, WT)
    isd_f = isd.reshape(N)
    aggT = _sc_heavy(src, dst, w_out, isd_f, xwT)
    xT, xws = _tc_postagg(aggT, xwT, invdeg, bcol, WpT)
    aggsp = _sc_score(src, dst, w_out, isd_f, xws.reshape(N))
    mask_out, xgT_out, xrT, kl = _tc_postscore23(
        aggsp, xws, invdeg, bp, mask_in, xT, batch_col, batch_row, k, n_level)
    return w_out, mask_out, xgT_out, xrT, kl


def kernel(istraining, fea, adj_index, adj_weight, batch_index, label,
           W1, b1, Wp1, bp1, W2, b2, Wp2, bp2, W3, b3, Wp3, bp3,
           Wl1, bl1, Wl2, bl2, Wl3, bl3):
    src, dst = adj_index[0], adj_index[1]
    src_i = src.astype(jnp.int32)
    dst_i = dst.astype(jnp.int32)

    xw = fea @ W1
    deg = jnp.zeros((N,), jnp.float32).at[dst].add(adj_weight) + 1.0
    degprod = _sc_degprod(src_i, dst_i, deg)
    norm1 = adj_weight / jnp.sqrt(degprod)
    upd = _sc_rowgather(src_i, xw) * norm1[:, None]
    agg = jnp.zeros_like(xw).at[dst].add(upd)
    x = jax.nn.relu(agg + xw / deg[:, None] + b1)
    xw_s = x @ Wp1
    us = _sc_smul(src_i, xw_s[:, 0], norm1)
    aggs = jnp.zeros((N, 1), jnp.float32).at[dst].add(us[:, None])
    s1 = (aggs + xw_s / deg[:, None] + bp1)[:, 0]
    _, perm1 = jax.lax.top_k(s1, 5000)

    xT = x.T
    batch_col = batch_index.reshape(N, 1).astype(jnp.int32)
    batch_row = batch_index.reshape(1, N).astype(jnp.int32)
    ones_row = jnp.ones((1, N), jnp.float32)

    mask1, xg1T, xr1T, kl1 = _tc_postscore1(
        s1.reshape(1, N), ones_row, xT, batch_col, batch_row, 5000, N)

    w2, mask2, xg2T, xr2T, kl2 = _level23(
        src_i, dst_i, adj_weight, mask1, xg1T, W2.T, b2.reshape(NHID, 1),
        Wp2.T, bp2.reshape(1, 1), batch_col, batch_row, 2500, 5000)
    _, _, _, xr3T, kl3 = _level23(
        src_i, dst_i, w2, mask2, xg2T, W3.T, b3.reshape(NHID, 1),
        Wp3.T, bp3.reshape(1, 1), batch_col, batch_row, 1250, 2500)

    outT, loss = _tc_head(
        xr1T, xr2T, xr3T, label[:, 0:1].reshape(1, NG),
        Wl1.T, bl1.reshape(NHID, 1), Wl2.T, bl2.reshape(NHID // 2, 1),
        Wl3.T, bl3.reshape(1, 1))

    kl_all = (kl1 + kl2 + kl3)[0, 0]
    return outT.reshape(NG, 1), loss[0, 0], kl_all, perm1


# speedup vs baseline: 1.0796x; 1.0796x over previous
import functools

import jax
import jax.numpy as jnp
from jax import lax
from jax.experimental import pallas as pl
from jax.experimental.pallas import tpu as pltpu
from jax.experimental.pallas import tpu_sc as plsc

NHID = 128
NG = 64
N = 10000
E = 320000
NWORK = 32
SHARD = E // NWORK
FPT = NHID // NWORK
WIN = 4000

_SC_MESH = plsc.VectorSubcoreMesh(core_axis_name="c", subcore_axis_name="s",
                                  num_cores=2, num_subcores=16)
_SC_PARAMS = pltpu.CompilerParams(needs_layout_passes=False)


def _wid():
    return lax.axis_index("s") * 2 + lax.axis_index("c")


def _gcn(x, W, b, src, dst, w):
    n = x.shape[0]
    xw = x @ W
    deg = jnp.zeros((n,), x.dtype).at[dst].add(w) + 1.0
    norm = w / jnp.sqrt(deg[src] * deg[dst])
    agg = jnp.zeros_like(xw).at[dst].add(xw[src] * norm[:, None])
    agg = agg + xw / deg[:, None]
    return agg + b


def _tc_pre_kernel(degp_ref, xgT_ref, WT_ref, xwT_ref, isd_ref, invdeg_ref):
    deg = jnp.sum(degp_ref[...], axis=0, keepdims=True) + 1.0
    isd_ref[...] = lax.rsqrt(deg)
    invdeg_ref[...] = 1.0 / deg
    xwT_ref[...] = jnp.dot(WT_ref[...], xgT_ref[...],
                           preferred_element_type=jnp.float32)


def _tc_pre(degp, xgT, WT):
    return pl.pallas_call(
        _tc_pre_kernel,
        name="tc_pre",
        out_shape=(
            jax.ShapeDtypeStruct((NHID, N), jnp.float32),
            jax.ShapeDtypeStruct((1, N), jnp.float32),
            jax.ShapeDtypeStruct((1, N), jnp.float32),
        ),
    )(degp, xgT, WT)


def _tc_postagg_kernel(aggT_ref, xwT_ref, invdeg_ref, bcol_ref, WpT_ref,
                       xT_ref, xws_ref):
    xT = jnp.maximum(
        aggT_ref[...] + xwT_ref[...] * invdeg_ref[...] + bcol_ref[...], 0.0)
    xT_ref[...] = xT
    xws_ref[...] = jnp.dot(WpT_ref[...], xT,
                           preferred_element_type=jnp.float32)


def _tc_postagg(aggT, xwT, invdeg, bcol, WpT):
    return pl.pallas_call(
        _tc_postagg_kernel,
        name="tc_postagg",
        out_shape=(
            jax.ShapeDtypeStruct((NHID, N), jnp.float32),
            jax.ShapeDtypeStruct((1, N), jnp.float32),
        ),
    )(aggT, xwT, invdeg, bcol, WpT)


def _postscore_body(s, mask_in_ref, xT_ref, batch_col_ref, batch_row_ref,
                    k, n_level, mask_ref, xgT_ref, xrT_ref, kl_ref):
    kept = mask_in_ref[...] > 0.0

    neg_inf = jnp.float32(-jnp.inf)
    smax = jnp.max(jnp.where(kept, s, neg_inf))
    e = jnp.where(kept, jnp.exp(s - smax), 0.0)
    p = e / jnp.sum(e)
    term = p * (jnp.log(p + 1e-12) + jnp.log(jnp.float32(n_level)))
    kl_ref[...] = jnp.sum(jnp.where(kept, term, 0.0)).reshape(1, 1)

    ubits = lax.bitcast_convert_type(s, jnp.uint32)
    u = jnp.where(s < 0.0, ~ubits, ubits | jnp.uint32(0x80000000))
    u = jnp.where(kept, u, jnp.uint32(0))
    kf = jnp.float32(k)

    def sel_bit(i, P):
        cand = P | (jnp.uint32(1) << (jnp.uint32(31) - i.astype(jnp.uint32)))
        c = jnp.sum((u >= cand).astype(jnp.float32))
        return jnp.where(c >= kf, cand, P)

    P = lax.fori_loop(0, 32, sel_bit, jnp.uint32(0))
    c_strict = jnp.sum((u > P).astype(jnp.float32))
    need = kf - c_strict
    tie = kept & (u == P)
    iota = lax.broadcasted_iota(jnp.int32, (1, N), 1)

    def idx_bit(i, t):
        cand = t + (jnp.int32(1) << (jnp.int32(13) - i))
        c = jnp.sum((tie & (iota < cand)).astype(jnp.float32))
        return jnp.where(c < need, cand, t)

    t_acc = lax.fori_loop(0, 14, idx_bit, jnp.int32(0))
    sel = (u > P) | (tie & (iota <= t_acc) & (need > 0.5))
    mask_out = jnp.where(sel, 1.0, 0.0)
    mask_ref[...] = mask_out

    gate = jnp.tanh(s) * mask_out
    xgT = xT_ref[...] * gate
    xgT_ref[...] = xgT

    B = (batch_col_ref[...] == lax.broadcasted_iota(
        jnp.int32, (N, NG), 1)).astype(jnp.float32)
    ssumT = jnp.dot(xgT, B, preferred_element_type=jnp.float32)
    cnt = jnp.dot(mask_out, B, preferred_element_type=jnp.float32)
    meanT = ssumT / jnp.maximum(cnt, 1.0)

    batch_row = batch_row_ref[...]
    col_iota = lax.broadcasted_iota(jnp.int32, (1, NG), 1)

    def mx_body(g, mxT):
        selg = (batch_row == g) & sel
        cand = jnp.max(jnp.where(selg, xgT, neg_inf), axis=1, keepdims=True)
        cand_b = jnp.broadcast_to(cand, (NHID, NG))
        return jnp.where(col_iota == g, cand_b, mxT)

    mxT = lax.fori_loop(0, NG, mx_body, jnp.full((NHID, NG), neg_inf))
    mxT = jnp.where(mxT == neg_inf, 0.0, mxT)
    xrT_ref[...] = jnp.concatenate([mxT, meanT], axis=0)


def _tc_postscore1_kernel(s_ref, mask_in_ref, xT_ref, batch_col_ref,
                          batch_row_ref, mask_ref, xgT_ref, xrT_ref, kl_ref,
                          *, k, n_level):
    _postscore_body(s_ref[...], mask_in_ref, xT_ref, batch_col_ref,
                    batch_row_ref, k, n_level, mask_ref, xgT_ref, xrT_ref,
                    kl_ref)


def _tc_postscore23_kernel(aggsp_ref, xws_ref, invdeg_ref, bp_ref, mask_in_ref,
                           xT_ref, batch_col_ref, batch_row_ref, mask_ref,
                           xgT_ref, xrT_ref, kl_ref, *, k, n_level):
    s = (jnp.sum(aggsp_ref[...], axis=0, keepdims=True)
         + xws_ref[...] * invdeg_ref[...] + bp_ref[0, 0])
    _postscore_body(s, mask_in_ref, xT_ref, batch_col_ref, batch_row_ref,
                    k, n_level, mask_ref, xgT_ref, xrT_ref, kl_ref)


_POSTSCORE_OUT = (
    jax.ShapeDtypeStruct((1, N), jnp.float32),
    jax.ShapeDtypeStruct((NHID, N), jnp.float32),
    jax.ShapeDtypeStruct((2 * NHID, NG), jnp.float32),
    jax.ShapeDtypeStruct((1, 1), jnp.float32),
)


def _tc_postscore1(s, mask_in, xT, batch_col, batch_row, k, n_level):
    return pl.pallas_call(
        functools.partial(_tc_postscore1_kernel, k=k, n_level=n_level),
        name="tc_postscore1",
        out_shape=_POSTSCORE_OUT,
    )(s, mask_in, xT, batch_col, batch_row)


def _tc_postscore23(aggsp, xws, invdeg, bp, mask_in, xT, batch_col, batch_row,
                    k, n_level):
    return pl.pallas_call(
        functools.partial(_tc_postscore23_kernel, k=k, n_level=n_level),
        name="tc_postscore23",
        out_shape=_POSTSCORE_OUT,
    )(aggsp, xws, invdeg, bp, mask_in, xT, batch_col, batch_row)


def _tc_head_kernel(x1_ref, x2_ref, x3_ref, lab_ref, W1T_ref, b1c_ref,
                    W2T_ref, b2c_ref, W3T_ref, b3_ref, out_ref, loss_ref):
    xrT = x1_ref[...] + x2_ref[...] + x3_ref[...]
    h = jnp.maximum(
        jnp.dot(W1T_ref[...], xrT, preferred_element_type=jnp.float32)
        + b1c_ref[...], 0.0)
    h = jnp.maximum(
        jnp.dot(W2T_ref[...], h, preferred_element_type=jnp.float32)
        + b2c_ref[...], 0.0)
    outT = jnp.dot(W3T_ref[...], h,
                   preferred_element_type=jnp.float32) + b3_ref[0, 0]
    out_ref[...] = outT
    loss_ref[...] = jnp.mean(jnp.abs(outT - lab_ref[...])).reshape(1, 1)


def _tc_head(xrT1, xrT2, xrT3, labT, Wl1T, bl1c, Wl2T, bl2c, Wl3T, bl3):
    return pl.pallas_call(
        _tc_head_kernel,
        name="tc_head",
        out_shape=(
            jax.ShapeDtypeStruct((1, NG), jnp.float32),
            jax.ShapeDtypeStruct((1, 1), jnp.float32),
        ),
    )(xrT1, xrT2, xrT3, labT, Wl1T, bl1c, Wl2T, bl2c, Wl3T, bl3)


def _zero_vmem(ref, n):
    def body(i, _):
        ref[pl.ds(i * 16, 16)] = jnp.zeros((16,), jnp.float32)
        return 0
    lax.fori_loop(0, n // 16, body, 0)


def _sc_prep_body(src_hbm, dst_hbm, w_hbm, mask_hbm, wout_hbm, degp_hbm,
                  srcv, dstv, wv, maskv, woutv, accv):
    wid = _wid()
    base = wid * SHARD
    pltpu.sync_copy(src_hbm.at[pl.ds(base, SHARD)], srcv)
    pltpu.sync_copy(dst_hbm.at[pl.ds(base, SHARD)], dstv)
    pltpu.sync_copy(w_hbm.at[pl.ds(base, SHARD)], wv)
    pltpu.sync_copy(mask_hbm, maskv)
    _zero_vmem(accv, N)

    @plsc.parallel_loop(0, SHARD // 16, unroll=4)
    def edge(g):
        sl = pl.ds(g * 16, 16)
        s16, d16, w16 = srcv[sl], dstv[sl], wv[sl]
        wo = (w16 * plsc.load_gather(maskv, [s16])
              * plsc.load_gather(maskv, [d16]))
        woutv[sl] = wo
        plsc.addupdate_scatter(accv, [d16], wo)
    pltpu.sync_copy(woutv, wout_hbm.at[pl.ds(base, SHARD)])
    pltpu.sync_copy(accv, degp_hbm.at[wid])


_sc_prep = functools.partial(
    pl.kernel, _sc_prep_body, name="sc_prep", mesh=_SC_MESH, compiler_params=_SC_PARAMS,
    out_type=(
        jax.ShapeDtypeStruct((E,), jnp.float32),
        jax.ShapeDtypeStruct((NWORK, N), jnp.float32),
    ),
    scratch_types=[
        pltpu.VMEM((SHARD,), jnp.int32),
        pltpu.VMEM((SHARD,), jnp.int32),
        pltpu.VMEM((SHARD,), jnp.float32),
        pltpu.VMEM((N,), jnp.float32),
        pltpu.VMEM((SHARD,), jnp.float32),
        pltpu.VMEM((N,), jnp.float32),
    ],
)()


def _sc_score_body(src_hbm, dst_hbm, w_hbm, isd_hbm, xws_hbm, aggsp_hbm,
                   srcv, dstv, wv, isdv, xwsv, accv):
    wid = _wid()
    base = wid * SHARD
    pltpu.sync_copy(src_hbm.at[pl.ds(base, SHARD)], srcv)
    pltpu.sync_copy(dst_hbm.at[pl.ds(base, SHARD)], dstv)
    pltpu.sync_copy(w_hbm.at[pl.ds(base, SHARD)], wv)
    pltpu.sync_copy(isd_hbm, isdv)
    pltpu.sync_copy(xws_hbm, xwsv)
    _zero_vmem(accv, N)

    @plsc.parallel_loop(0, SHARD // 16, unroll=4)
    def edge(g):
        sl = pl.ds(g * 16, 16)
        s16, d16, w16 = srcv[sl], dstv[sl], wv[sl]
        n16 = (w16 * plsc.load_gather(isdv, [s16])
               * plsc.load_gather(isdv, [d16]))
        val = plsc.load_gather(xwsv, [s16]) * n16
        plsc.addupdate_scatter(accv, [d16], val)
    pltpu.sync_copy(accv, aggsp_hbm.at[wid])


_sc_score = functools.partial(
    pl.kernel, _sc_score_body, name="sc_score", mesh=_SC_MESH, compiler_params=_SC_PARAMS,
    out_type=jax.ShapeDtypeStruct((NWORK, N), jnp.float32),
    scratch_types=[
        pltpu.VMEM((SHARD,), jnp.int32),
        pltpu.VMEM((SHARD,), jnp.int32),
        pltpu.VMEM((SHARD,), jnp.float32),
        pltpu.VMEM((N,), jnp.float32),
        pltpu.VMEM((N,), jnp.float32),
        pltpu.VMEM((N,), jnp.float32),
    ],
)()


def _sc_heavy_body(src_hbm, dst_hbm, w_hbm, isd_hbm, xwTf_hbm, aggTf_hbm,
                   srcwv, dstwv, wwv, isdv, *frefs):
    xwfs, accfs = frefs[:FPT], frefs[FPT:]
    wid = _wid()
    f0 = wid * FPT
    for f in range(FPT):
        pltpu.sync_copy(xwTf_hbm.at[pl.ds((f0 + f) * N, N)], xwfs[f])
        _zero_vmem(accfs[f], N)
    pltpu.sync_copy(isd_hbm, isdv)

    def window(wi, _):
        b = wi * WIN
        pltpu.sync_copy(src_hbm.at[pl.ds(b, WIN)], srcwv)
        pltpu.sync_copy(dst_hbm.at[pl.ds(b, WIN)], dstwv)
        pltpu.sync_copy(w_hbm.at[pl.ds(b, WIN)], wwv)

        @plsc.parallel_loop(0, WIN // 16, unroll=8)
        def edge(g):
            sl = pl.ds(g * 16, 16)
            s16, d16, w16 = srcwv[sl], dstwv[sl], wwv[sl]
            n16 = (w16 * plsc.load_gather(isdv, [s16])
                   * plsc.load_gather(isdv, [d16]))
            for f in range(FPT):
                v = plsc.load_gather(xwfs[f], [s16]) * n16
                plsc.addupdate_scatter(accfs[f], [d16], v)

        return 0

    lax.fori_loop(0, E // WIN, window, 0)
    for f in range(FPT):
        pltpu.sync_copy(accfs[f], aggTf_hbm.at[pl.ds((f0 + f) * N, N)])


_sc_heavy_flat = functools.partial(
    pl.kernel, _sc_heavy_body, name="sc_heavy", mesh=_SC_MESH, compiler_params=_SC_PARAMS,
    out_type=jax.ShapeDtypeStruct((NHID * N,), jnp.float32),
    scratch_types=[
        pltpu.VMEM((WIN,), jnp.int32),
        pltpu.VMEM((WIN,), jnp.int32),
        pltpu.VMEM((WIN,), jnp.float32),
        pltpu.VMEM((N,), jnp.float32),
    ] + [pltpu.VMEM((N,), jnp.float32)] * (2 * FPT),
)()


def _sc_heavy(src, dst, w, isd, xwT):
    aggTf = _sc_heavy_flat(src, dst, w, isd, xwT.reshape(NHID * N))
    return aggTf.reshape(NHID, N)


CH = 80


def _sc_degprod_body(src_hbm, dst_hbm, deg_hbm, out_hbm, srcv, dstv, degv,
                     outv):
    wid = _wid()
    base = wid * SHARD
    pltpu.sync_copy(src_hbm.at[pl.ds(base, SHARD)], srcv)
    pltpu.sync_copy(dst_hbm.at[pl.ds(base, SHARD)], dstv)
    pltpu.sync_copy(deg_hbm, degv)

    @plsc.parallel_loop(0, SHARD // 16, unroll=4)
    def edge(g):
        sl = pl.ds(g * 16, 16)
        outv[sl] = (plsc.load_gather(degv, [srcv[sl]])
                    * plsc.load_gather(degv, [dstv[sl]]))
    pltpu.sync_copy(outv, out_hbm.at[pl.ds(base, SHARD)])


_sc_degprod = functools.partial(
    pl.kernel, _sc_degprod_body, name="sc_degprod", mesh=_SC_MESH, compiler_params=_SC_PARAMS,
    out_type=jax.ShapeDtypeStruct((E,), jnp.float32),
    scratch_types=[
        pltpu.VMEM((SHARD,), jnp.int32),
        pltpu.VMEM((SHARD,), jnp.int32),
        pltpu.VMEM((N,), jnp.float32),
        pltpu.VMEM((SHARD,), jnp.float32),
    ],
)()


def _sc_smul_body(src_hbm, xws_hbm, norm_hbm, out_hbm, srcv, xwsv, normv,
                  outv):
    wid = _wid()
    base = wid * SHARD
    pltpu.sync_copy(src_hbm.at[pl.ds(base, SHARD)], srcv)
    pltpu.sync_copy(norm_hbm.at[pl.ds(base, SHARD)], normv)
    pltpu.sync_copy(xws_hbm, xwsv)

    @plsc.parallel_loop(0, SHARD // 16, unroll=4)
    def edge(g):
        sl = pl.ds(g * 16, 16)
        outv[sl] = plsc.load_gather(xwsv, [srcv[sl]]) * normv[sl]
    pltpu.sync_copy(outv, out_hbm.at[pl.ds(base, SHARD)])


_sc_smul = functools.partial(
    pl.kernel, _sc_smul_body, name="sc_smul", mesh=_SC_MESH, compiler_params=_SC_PARAMS,
    out_type=jax.ShapeDtypeStruct((E,), jnp.float32),
    scratch_types=[
        pltpu.VMEM((SHARD,), jnp.int32),
        pltpu.VMEM((N,), jnp.float32),
        pltpu.VMEM((SHARD,), jnp.float32),
        pltpu.VMEM((SHARD,), jnp.float32),
    ],
)()


def _sc_rowgather_body(src_hbm, xw_hbm, out_hbm, srcc, rows, sem):
    wid = _wid()
    base = wid * SHARD

    def chunk(c, _):
        b = base + c * CH
        pltpu.sync_copy(src_hbm.at[pl.ds(b, CH)], srcc)
        pltpu.async_copy(xw_hbm.at[srcc], rows, sem).wait()
        pltpu.sync_copy(rows, out_hbm.at[pl.ds(b, CH)])
        return 0

    lax.fori_loop(0, SHARD // CH, chunk, 0)


_sc_rowgather = functools.partial(
    pl.kernel, _sc_rowgather_body, name="sc_rowgather", mesh=_SC_MESH, compiler_params=_SC_PARAMS,
    out_type=jax.ShapeDtypeStruct((E, NHID), jnp.float32),
    scratch_types=[
        pltpu.VMEM((CH,), jnp.int32),
        pltpu.VMEM((CH, NHID), jnp.float32),
        pltpu.SemaphoreType.DMA,
    ],
)()


def _level23(src, dst, w_in, mask_in, xgT, WT, bcol, WpT, bp, batch_col,
             batch_row, k, n_level):
    w_out, degp = _sc_prep(src, dst, w_in, mask_in.reshape(N))
    xwT, isd, invdeg = _tc_pre(degp, xgT, WT)
    isd_f = isd.reshape(N)
    aggT = _sc_heavy(src, dst, w_out, isd_f, xwT)
    xT, xws = _tc_postagg(aggT, xwT, invdeg, bcol, WpT)
    aggsp = _sc_score(src, dst, w_out, isd_f, xws.reshape(N))
    mask_out, xgT_out, xrT, kl = _tc_postscore23(
        aggsp, xws, invdeg, bp, mask_in, xT, batch_col, batch_row, k, n_level)
    return w_out, mask_out, xgT_out, xrT, kl


def kernel(istraining, fea, adj_index, adj_weight, batch_index, label,
           W1, b1, Wp1, bp1, W2, b2, Wp2, bp2, W3, b3, Wp3, bp3,
           Wl1, bl1, Wl2, bl2, Wl3, bl3):
    src, dst = adj_index[0], adj_index[1]
    src_i = src.astype(jnp.int32)
    dst_i = dst.astype(jnp.int32)

    xw = fea @ W1
    deg = jnp.zeros((N,), jnp.float32).at[dst].add(adj_weight) + 1.0
    degprod = _sc_degprod(src_i, dst_i, deg)
    norm1 = adj_weight / jnp.sqrt(degprod)
    upd = _sc_rowgather(src_i, xw) * norm1[:, None]
    agg = jnp.zeros_like(xw).at[dst].add(upd)
    x = jax.nn.relu(agg + xw / deg[:, None] + b1)
    xw_s = x @ Wp1
    us = _sc_smul(src_i, xw_s[:, 0], norm1)
    aggs = jnp.zeros((N, 1), jnp.float32).at[dst].add(us[:, None])
    s1 = (aggs + xw_s / deg[:, None] + bp1)[:, 0]
    _, perm1 = jax.lax.top_k(s1, 5000)

    xT = x.T
    batch_col = batch_index.reshape(N, 1).astype(jnp.int32)
    batch_row = batch_index.reshape(1, N).astype(jnp.int32)
    ones_row = jnp.ones((1, N), jnp.float32)

    mask1, xg1T, xr1T, kl1 = _tc_postscore1(
        s1.reshape(1, N), ones_row, xT, batch_col, batch_row, 5000, N)

    w2, mask2, xg2T, xr2T, kl2 = _level23(
        src_i, dst_i, adj_weight, mask1, xg1T, W2.T, b2.reshape(NHID, 1),
        Wp2.T, bp2.reshape(1, 1), batch_col, batch_row, 2500, 5000)
    _, _, _, xr3T, kl3 = _level23(
        src_i, dst_i, w2, mask2, xg2T, W3.T, b3.reshape(NHID, 1),
        Wp3.T, bp3.reshape(1, 1), batch_col, batch_row, 1250, 2500)

    outT, loss = _tc_head(
        xr1T, xr2T, xr3T, label[:, 0:1].reshape(1, NG),
        Wl1.T, bl1.reshape(NHID, 1), Wl2.T, bl2.reshape(NHID // 2, 1),
        Wl3.T, bl3.reshape(1, 1))

    kl_all = (kl1 + kl2 + kl3)[0, 0]
    return outT.reshape(NG, 1), loss[0, 0], kl_all, perm1
